# baseline (speedup 1.0000x reference)
import jax
import jax.numpy as jnp
from jax.experimental import pallas as pl
from jax.experimental.pallas import tpu as pltpu

TOKEN_BLOCK = 512


def _moe_dense_body(idx_ref, prob_ref, x_ref, w_ref, b_ref, out_ref):
    e = pl.program_id(1)
    idx = idx_ref[...]
    p = prob_ref[...]
    gate = jnp.sum(jnp.where(idx == e, p, 0.0), axis=1)
    y = jnp.dot(x_ref[...].astype(jnp.bfloat16), w_ref[0].astype(jnp.bfloat16),
                preferred_element_type=jnp.float32)
    y = y + b_ref[0]
    contrib = gate[:, None] * y

    @pl.when(e == 0)
    def _init():
        out_ref[...] = contrib

    @pl.when(e > 0)
    def _acc():
        out_ref[...] += contrib


def kernel(input_batch, probabilities, indices, W, b):
    n_tokens, d_model = input_batch.shape
    n_experts, _, d_out = W.shape
    idx32 = indices.astype(jnp.int32)
    grid = (n_tokens // TOKEN_BLOCK, n_experts)
    out = pl.pallas_call(
        _moe_dense_body,
        grid=grid,
        in_specs=[
            pl.BlockSpec((TOKEN_BLOCK, idx32.shape[1]), lambda t, e: (t, 0)),
            pl.BlockSpec((TOKEN_BLOCK, probabilities.shape[1]), lambda t, e: (t, 0)),
            pl.BlockSpec((TOKEN_BLOCK, d_model), lambda t, e: (t, 0)),
            pl.BlockSpec((1, d_model, d_out), lambda t, e: (e, 0, 0)),
            pl.BlockSpec((1, 1, d_out), lambda t, e: (e, 0, 0)),
        ],
        out_specs=pl.BlockSpec((TOKEN_BLOCK, d_out), lambda t, e: (t, 0)),
        out_shape=jax.ShapeDtypeStruct((n_tokens, d_out), input_batch.dtype),
    )(idx32, probabilities, input_batch, W, b.reshape(n_experts, 1, d_out))
    total_loss = jnp.asarray(0.0, dtype=jnp.float32)
    return (out, total_loss)


# speedup vs baseline: 1.7397x; 1.7397x over previous
import jax
import jax.numpy as jnp
from jax.experimental import pallas as pl
from jax.experimental.pallas import tpu as pltpu

TOKEN_BLOCK = 512


def _moe_dense_body(idx_ref, prob_ref, x_ref, w_ref, b_ref, out_ref):
    idx = idx_ref[...]
    p = prob_ref[...]
    x = x_ref[...].astype(jnp.bfloat16)
    n_experts = w_ref.shape[0]
    acc = None
    for e in range(n_experts):
        gate = jnp.sum(jnp.where(idx == e, p, 0.0), axis=1)
        y = jnp.dot(x, w_ref[e].astype(jnp.bfloat16),
                    preferred_element_type=jnp.float32)
        y = y + b_ref[e]
        contrib = gate[:, None] * y
        acc = contrib if acc is None else acc + contrib
    out_ref[...] = acc


def kernel(input_batch, probabilities, indices, W, b):
    n_tokens, d_model = input_batch.shape
    n_experts, _, d_out = W.shape
    idx32 = indices.astype(jnp.int32)
    grid = (n_tokens // TOKEN_BLOCK,)
    out = pl.pallas_call(
        _moe_dense_body,
        grid=grid,
        in_specs=[
            pl.BlockSpec((TOKEN_BLOCK, idx32.shape[1]), lambda t: (t, 0)),
            pl.BlockSpec((TOKEN_BLOCK, probabilities.shape[1]), lambda t: (t, 0)),
            pl.BlockSpec((TOKEN_BLOCK, d_model), lambda t: (t, 0)),
            pl.BlockSpec((n_experts, d_model, d_out), lambda t: (0, 0, 0)),
            pl.BlockSpec((n_experts, 1, d_out), lambda t: (0, 0, 0)),
        ],
        out_specs=pl.BlockSpec((TOKEN_BLOCK, d_out), lambda t: (t, 0)),
        out_shape=jax.ShapeDtypeStruct((n_tokens, d_out), input_batch.dtype),
    )(idx32, probabilities, input_batch, W, b.reshape(n_experts, 1, d_out))
    total_loss = jnp.asarray(0.0, dtype=jnp.float32)
    return (out, total_loss)
